# Initial kernel scaffold; baseline (speedup 1.0000x reference)
import jax
import jax.numpy as jnp
from jax.experimental import pallas as pl


def kernel(x, edge_index, edge_weight, W1, W2):
    raise NotImplementedError("write your pallas kernel here")


# speedup vs baseline: 1.9439x; 1.9439x over previous
import functools

import jax
import jax.numpy as jnp
from jax import lax
from jax.experimental import pallas as pl
from jax.experimental.pallas import tpu as pltpu
from jax.experimental.pallas import tpu_sc as plsc

N_NODES = 10000
N_EDGES = 160000
D_IN = 256
D_H1 = 256
D_H2 = 64

NC = 2
NS = 16
L = 16
FW = 128
NPAD = 10240


def _mm_body(x_ref, w_ref, o_ref):
    o_ref[...] = jnp.dot(x_ref[...], w_ref[...],
                         preferred_element_type=jnp.float32)


def _tc_matmul(x, w, blk=1000):
    n, k = x.shape
    m = w.shape[1]
    return pl.pallas_call(
        _mm_body,
        grid=(n // blk,),
        in_specs=[
            pl.BlockSpec((blk, k), lambda i: (i, 0)),
            pl.BlockSpec((k, m), lambda i: (0, 0)),
        ],
        out_specs=pl.BlockSpec((blk, m), lambda i: (i, 0)),
        out_shape=jax.ShapeDtypeStruct((n, m), jnp.float32),
    )(x, w)


def _relu_mm_body(h_ref, w_ref, o_ref):
    a = jnp.maximum(h_ref[0], 0.0)
    b = jnp.maximum(h_ref[1], 0.0)
    o_ref[...] = (jnp.dot(a, w_ref[0], preferred_element_type=jnp.float32)
                  + jnp.dot(b, w_ref[1], preferred_element_type=jnp.float32))


def _tc_relu_matmul(h_split, w_split, blk=1024):
    _, n, k = h_split.shape
    m = w_split.shape[2]
    return pl.pallas_call(
        _relu_mm_body,
        grid=(n // blk,),
        in_specs=[
            pl.BlockSpec((2, blk, k), lambda i: (0, i, 0)),
            pl.BlockSpec((2, k, m), lambda i: (0, 0, 0)),
        ],
        out_specs=pl.BlockSpec((blk, m), lambda i: (i, 0)),
        out_shape=jax.ShapeDtypeStruct((n, m), jnp.float32),
    )(h_split, w_split)


def _decoder_body(a0_ref, a1_ref, b0_ref, b1_ref, o_ref):
    a = a0_ref[...] + a1_ref[...]
    b = b0_ref[...] + b1_ref[...]
    o_ref[...] = lax.dot_general(
        a, b, (((1,), (1,)), ((), ())),
        preferred_element_type=jnp.float32)


def _tc_decoder(z0, z1, rblk=200):
    npad, k = z0.shape
    n = N_NODES
    return pl.pallas_call(
        _decoder_body,
        grid=(n // rblk,),
        in_specs=[
            pl.BlockSpec((rblk, k), lambda i: (i, 0)),
            pl.BlockSpec((rblk, k), lambda i: (i, 0)),
            pl.BlockSpec((n, k), lambda i: (0, 0)),
            pl.BlockSpec((n, k), lambda i: (0, 0)),
        ],
        out_specs=pl.BlockSpec((rblk, n), lambda i: (i, 0)),
        out_shape=jax.ShapeDtypeStruct((n, n), jnp.float32),
    )(z0, z1, z0, z1)


def _segsum_body(table, src_h, dst_h, ew_h, zero_h, out,
                 acc, src_v, dst_v, ew_v, rows_v,
                 *, edge_split, chunk, scale_vecs):
    c = lax.axis_index("c")
    s = lax.axis_index("s")
    npt = NPAD // NS

    pltpu.sync_copy(zero_h, acc.at[pl.ds(s * npt, npt)])
    plsc.subcore_barrier()

    if edge_split:
        ept = N_EDGES // (NC * NS)
        ebase = (c * NS + s) * ept
        tbl = table
    else:
        ept = N_EDGES // NS
        ebase = s * ept
        tbl = table.at[c]
    nch = ept // chunk

    def scale_group(base_e, wv):
        for j in range(L):
            e = base_e + j
            w = jnp.full((L,), wv[j], jnp.float32)
            for f in range(scale_vecs):
                rows_v[e, pl.ds(f * L, L)] = (
                    rows_v[e, pl.ds(f * L, L)] * w)

    def chunk_step(k, carry):
        base = ebase + k * chunk
        pltpu.sync_copy(src_h.at[pl.ds(base, chunk)], src_v)
        pltpu.sync_copy(dst_h.at[pl.ds(base, chunk)], dst_v)
        pltpu.sync_copy(ew_h.at[pl.ds(base, chunk)], ew_v.at[pl.ds(0, chunk)])
        pltpu.sync_copy(tbl.at[src_v], rows_v)

        full_groups = chunk // L
        rem = chunk % L

        def group_step(g, carry2):
            scale_group(g * L, ew_v[pl.ds(g * L, L)])
            return carry2

        lax.fori_loop(0, full_groups, group_step, 0)
        if rem:
            wv = ew_v[pl.ds(full_groups * L, L)]
            for j in range(rem):
                e = full_groups * L + j
                w = jnp.full((L,), wv[j], jnp.float32)
                for f in range(scale_vecs):
                    rows_v[e, pl.ds(f * L, L)] = (
                        rows_v[e, pl.ds(f * L, L)] * w)
        pltpu.sync_copy(rows_v, acc.at[dst_v], add=True)
        return carry

    lax.fori_loop(0, nch, chunk_step, 0)
    plsc.subcore_barrier()

    pltpu.sync_copy(acc.at[pl.ds(s * npt, npt)],
                    out.at[c].at[pl.ds(s * npt, npt)])


def _make_segsum(table_shape, *, edge_split, chunk, scale_vecs, name):
    body = functools.partial(
        _segsum_body, edge_split=edge_split, chunk=chunk,
        scale_vecs=scale_vecs)
    mesh = plsc.VectorSubcoreMesh(core_axis_name="c", subcore_axis_name="s")
    return pl.kernel(
        body,
        out_type=jax.ShapeDtypeStruct((NC, NPAD, FW), jnp.float32),
        mesh=mesh,
        scratch_types=[
            pltpu.VMEM_SHARED((NPAD, FW), jnp.float32),
            pltpu.VMEM((chunk,), jnp.int32),
            pltpu.VMEM((chunk,), jnp.int32),
            pltpu.VMEM((((chunk + L - 1) // L) * L,), jnp.float32),
            pltpu.VMEM((chunk, FW), jnp.float32),
        ],
        name=name,
    )


_segsum_l1 = _make_segsum((NC, N_NODES, FW), edge_split=False, chunk=80,
                          scale_vecs=FW // L, name="segsum_l1")
_segsum_l2 = _make_segsum((NPAD, FW), edge_split=True, chunk=40,
                          scale_vecs=D_H2 // L, name="segsum_l2")


@jax.jit
def kernel(x, edge_index, edge_weight, W1, W2):
    src = edge_index[0]
    dst = edge_index[1]
    zeros = jnp.zeros((NPAD // NS, FW), jnp.float32)

    hw1 = _tc_matmul(x, W1)
    hw1_split = hw1.reshape(N_NODES, NC, D_H1 // NC).transpose(1, 0, 2)
    h1_pre = _segsum_l1(hw1_split, src, dst, edge_weight, zeros)

    w2_split = jnp.pad(W2.reshape(NC, D_H1 // NC, D_H2),
                       ((0, 0), (0, 0), (0, FW - D_H2)))
    hz = _tc_relu_matmul(h1_pre, w2_split)
    z_parts = _segsum_l2(hz, src, dst, edge_weight, zeros)

    recon = _tc_decoder(z_parts[0], z_parts[1])
    return recon.reshape(-1)


# speedup vs baseline: 2.3545x; 1.2112x over previous
import functools

import jax
import jax.numpy as jnp
from jax import lax
from jax.experimental import pallas as pl
from jax.experimental.pallas import tpu as pltpu
from jax.experimental.pallas import tpu_sc as plsc

N_NODES = 10000
N_EDGES = 160000
D_IN = 256
D_H1 = 256
D_H2 = 64

NC = 2
NS = 16
L = 16
FW = 128
NPAD = 10240


def _mm_body(x_ref, w_ref, o_ref):
    o_ref[...] = jnp.dot(x_ref[...], w_ref[...],
                         preferred_element_type=jnp.float32)


def _tc_matmul(x, w, blk=1000):
    n, k = x.shape
    m = w.shape[1]
    return pl.pallas_call(
        _mm_body,
        grid=(n // blk,),
        in_specs=[
            pl.BlockSpec((blk, k), lambda i: (i, 0)),
            pl.BlockSpec((k, m), lambda i: (0, 0)),
        ],
        out_specs=pl.BlockSpec((blk, m), lambda i: (i, 0)),
        out_shape=jax.ShapeDtypeStruct((n, m), jnp.float32),
    )(x, w)


def _relu_mm_body(h_ref, w_ref, o_ref):
    a = jnp.maximum(h_ref[0], 0.0)
    b = jnp.maximum(h_ref[1], 0.0)
    o_ref[...] = (jnp.dot(a, w_ref[0], preferred_element_type=jnp.float32)
                  + jnp.dot(b, w_ref[1], preferred_element_type=jnp.float32))


def _tc_relu_matmul(h_split, w_split, blk=1024):
    _, n, k = h_split.shape
    m = w_split.shape[2]
    return pl.pallas_call(
        _relu_mm_body,
        grid=(n // blk,),
        in_specs=[
            pl.BlockSpec((2, blk, k), lambda i: (0, i, 0)),
            pl.BlockSpec((2, k, m), lambda i: (0, 0, 0)),
        ],
        out_specs=pl.BlockSpec((blk, m), lambda i: (i, 0)),
        out_shape=jax.ShapeDtypeStruct((n, m), jnp.float32),
    )(h_split, w_split)


def _decoder_body(a0_ref, a1_ref, b0_ref, b1_ref, o_ref):
    a = a0_ref[...] + a1_ref[...]
    b = b0_ref[...] + b1_ref[...]
    o_ref[...] = lax.dot_general(
        a, b, (((1,), (1,)), ((), ())),
        preferred_element_type=jnp.float32)


def _tc_decoder(z0, z1, rblk=200):
    npad, k = z0.shape
    n = N_NODES
    return pl.pallas_call(
        _decoder_body,
        grid=(n // rblk,),
        in_specs=[
            pl.BlockSpec((rblk, k), lambda i: (i, 0)),
            pl.BlockSpec((rblk, k), lambda i: (i, 0)),
            pl.BlockSpec((n, k), lambda i: (0, 0)),
            pl.BlockSpec((n, k), lambda i: (0, 0)),
        ],
        out_specs=pl.BlockSpec((rblk, n), lambda i: (i, 0)),
        out_shape=jax.ShapeDtypeStruct((n, n), jnp.float32),
    )(z0, z1, z0, z1)


CHUNK = 96
NROWS = 3
NIDX = 4


def _segsum_body(table, src_h, dst_h, ew_h, zero_h, out,
                 acc, srcv, dstv, ewv, rows, src_t, dst_t, ew_t, rows_t,
                 sem_src, sem_dst, sem_ew, sem_g, sem_s,
                 *, edge_split, scale_vecs):
    c = lax.axis_index("c")
    s = lax.axis_index("s")
    npt = NPAD // NS
    C = CHUNK

    if edge_split:
        ept = N_EDGES // (NC * NS)
        ebase = (c * NS + s) * ept
        tbl = table
    else:
        ept = N_EDGES // NS
        ebase = s * ept
        tbl = table.at[c]
    nch = ept // C
    tail = ept - nch * C

    pltpu.sync_copy(zero_h, acc.at[pl.ds(s * npt, npt)])
    plsc.subcore_barrier()

    def issue_idx(k):
        buf = lax.rem(k, NIDX)
        pltpu.async_copy(src_h.at[pl.ds(ebase + k * C, C)],
                         srcv.at[buf], sem_src.at[buf])
        pltpu.async_copy(dst_h.at[pl.ds(ebase + k * C, C)],
                         dstv.at[buf], sem_dst.at[buf])
        pltpu.async_copy(ew_h.at[pl.ds(ebase + k * C, C)],
                         ewv.at[buf], sem_ew.at[buf])

    def wait_src(k):
        buf = lax.rem(k, NIDX)
        pltpu.make_async_copy(src_h.at[pl.ds(ebase, C)],
                              srcv.at[buf], sem_src.at[buf]).wait()

    def wait_dst(k):
        buf = lax.rem(k, NIDX)
        pltpu.make_async_copy(dst_h.at[pl.ds(ebase, C)],
                              dstv.at[buf], sem_dst.at[buf]).wait()

    def wait_ew(k):
        buf = lax.rem(k, NIDX)
        pltpu.make_async_copy(ew_h.at[pl.ds(ebase, C)],
                              ewv.at[buf], sem_ew.at[buf]).wait()

    def issue_gather(k):
        rb = lax.rem(k, NROWS)
        ib = lax.rem(k, NIDX)
        pltpu.async_copy(tbl.at[srcv.at[ib]], rows.at[rb], sem_g.at[rb])

    def wait_gather(k):
        rb = lax.rem(k, NROWS)
        pltpu.make_async_copy(tbl.at[srcv.at[0]], rows.at[rb],
                              sem_g.at[rb]).wait()

    def issue_scatter(k):
        rb = lax.rem(k, NROWS)
        ib = lax.rem(k, NIDX)
        pltpu.async_copy(rows.at[rb], acc.at[dstv.at[ib]],
                         sem_s.at[rb], add=True)

    def wait_scatter(k):
        rb = lax.rem(k, NROWS)
        ib = lax.rem(k, NIDX)
        pltpu.make_async_copy(rows.at[rb], acc.at[dstv.at[ib]],
                              sem_s.at[rb]).wait()

    for kk in range(min(3, nch)):
        issue_idx(kk)
    if nch > 0:
        wait_src(0)
        issue_gather(0)
    if nch > 1:
        wait_src(1)
        issue_gather(1)

    def step(k, carry):
        ib = lax.rem(k, NIDX)
        rb = lax.rem(k, NROWS)
        wait_gather(k)

        @pl.when(k >= 1)
        def _drain():
            wait_scatter(k - 1)

        @pl.when(k + 3 < nch)
        def _prefetch_idx():
            issue_idx(k + 3)

        @pl.when(k + 2 < nch)
        def _prefetch_rows():
            wait_src(k + 2)
            issue_gather(k + 2)

        wait_ew(k)

        def group(g, carry2):
            wv = ewv[ib, pl.ds(g * L, L)]
            for j in range(L):
                e = g * L + j
                w = jnp.full((L,), wv[j], jnp.float32)
                for f in range(scale_vecs):
                    sl = pl.ds(f * L, L)
                    rows[rb, e, sl] = rows[rb, e, sl] * w
            return carry2

        lax.fori_loop(0, C // L, group, 0)
        wait_dst(k)
        issue_scatter(k)
        return carry

    lax.fori_loop(0, nch, step, 0)

    if tail:
        tb = nch * C
        pltpu.sync_copy(src_h.at[pl.ds(ebase + tb, tail)], src_t)
        pltpu.sync_copy(dst_h.at[pl.ds(ebase + tb, tail)], dst_t)
        pltpu.sync_copy(ew_h.at[pl.ds(ebase + tb, tail)],
                        ew_t.at[pl.ds(0, tail)])
        pltpu.sync_copy(tbl.at[src_t], rows_t)
        wv = ew_t[...]
        for j in range(tail):
            w = jnp.full((L,), wv[j], jnp.float32)
            for f in range(scale_vecs):
                sl = pl.ds(f * L, L)
                rows_t[j, sl] = rows_t[j, sl] * w
        pltpu.sync_copy(rows_t, acc.at[dst_t], add=True)

    wait_scatter(nch - 1)
    plsc.subcore_barrier()

    pltpu.sync_copy(acc.at[pl.ds(s * npt, npt)],
                    out.at[c].at[pl.ds(s * npt, npt)])


def _make_segsum(*, edge_split, scale_vecs, name):
    body = functools.partial(
        _segsum_body, edge_split=edge_split, scale_vecs=scale_vecs)
    mesh = plsc.VectorSubcoreMesh(core_axis_name="c", subcore_axis_name="s")
    ept = N_EDGES // (NC * NS) if edge_split else N_EDGES // NS
    tail = ept % CHUNK
    return pl.kernel(
        body,
        out_type=jax.ShapeDtypeStruct((NC, NPAD, FW), jnp.float32),
        mesh=mesh,
        scratch_types=[
            pltpu.VMEM_SHARED((NPAD, FW), jnp.float32),
            pltpu.VMEM((NIDX, CHUNK), jnp.int32),
            pltpu.VMEM((NIDX, CHUNK), jnp.int32),
            pltpu.VMEM((NIDX, CHUNK), jnp.float32),
            pltpu.VMEM((NROWS, CHUNK, FW), jnp.float32),
            pltpu.VMEM((max(tail, 1),), jnp.int32),
            pltpu.VMEM((max(tail, 1),), jnp.int32),
            pltpu.VMEM((L,), jnp.float32),
            pltpu.VMEM((max(tail, 1), FW), jnp.float32),
            pltpu.SemaphoreType.DMA((NIDX,)),
            pltpu.SemaphoreType.DMA((NIDX,)),
            pltpu.SemaphoreType.DMA((NIDX,)),
            pltpu.SemaphoreType.DMA((NROWS,)),
            pltpu.SemaphoreType.DMA((NROWS,)),
        ],
        name=name,
    )


_segsum_l1 = _make_segsum(edge_split=False, scale_vecs=FW // L,
                          name="segsum_l1")
_segsum_l2 = _make_segsum(edge_split=True, scale_vecs=D_H2 // L,
                          name="segsum_l2")


@jax.jit
def kernel(x, edge_index, edge_weight, W1, W2):
    src = edge_index[0]
    dst = edge_index[1]
    zeros = jnp.zeros((NPAD // NS, FW), jnp.float32)

    hw1 = _tc_matmul(x, W1)
    hw1_split = hw1.reshape(N_NODES, NC, D_H1 // NC).transpose(1, 0, 2)
    h1_pre = _segsum_l1(hw1_split, src, dst, edge_weight, zeros)

    w2_split = jnp.pad(W2.reshape(NC, D_H1 // NC, D_H2),
                       ((0, 0), (0, 0), (0, FW - D_H2)))
    hz = _tc_relu_matmul(h1_pre, w2_split)
    z_parts = _segsum_l2(hz, src, dst, edge_weight, zeros)

    recon = _tc_decoder(z_parts[0], z_parts[1])
    return recon.reshape(-1)


# speedup vs baseline: 2.7850x; 1.1828x over previous
import functools

import jax
import jax.numpy as jnp
from jax import lax
from jax.experimental import pallas as pl
from jax.experimental.pallas import tpu as pltpu
from jax.experimental.pallas import tpu_sc as plsc

N_NODES = 10000
N_EDGES = 160000
D_IN = 256
D_H1 = 256
D_H2 = 64

NC = 2
NS = 16
L = 16
FW = 128
NPAD = 10240


def _mm_body(x_ref, w_ref, o_ref):
    o_ref[...] = jnp.dot(x_ref[...], w_ref[...],
                         preferred_element_type=jnp.float32)


def _tc_matmul(x, w, blk=1000):
    n, k = x.shape
    m = w.shape[1]
    return pl.pallas_call(
        _mm_body,
        grid=(n // blk,),
        in_specs=[
            pl.BlockSpec((blk, k), lambda i: (i, 0)),
            pl.BlockSpec((k, m), lambda i: (0, 0)),
        ],
        out_specs=pl.BlockSpec((blk, m), lambda i: (i, 0)),
        out_shape=jax.ShapeDtypeStruct((n, m), jnp.float32),
    )(x, w)


def _relu_mm_body(h_ref, w_ref, o_ref):
    a = jnp.maximum(h_ref[0], 0.0)
    b = jnp.maximum(h_ref[1], 0.0)
    o_ref[...] = (jnp.dot(a, w_ref[0], preferred_element_type=jnp.float32)
                  + jnp.dot(b, w_ref[1], preferred_element_type=jnp.float32))


def _tc_relu_matmul(h_split, w_split, blk=1024):
    _, n, k = h_split.shape
    m = w_split.shape[2]
    return pl.pallas_call(
        _relu_mm_body,
        grid=(n // blk,),
        in_specs=[
            pl.BlockSpec((2, blk, k), lambda i: (0, i, 0)),
            pl.BlockSpec((2, k, m), lambda i: (0, 0, 0)),
        ],
        out_specs=pl.BlockSpec((blk, m), lambda i: (i, 0)),
        out_shape=jax.ShapeDtypeStruct((n, m), jnp.float32),
    )(h_split, w_split)


def _decoder_body(a0_ref, a1_ref, b0_ref, b1_ref, o_ref):
    a = a0_ref[...] + a1_ref[...]
    b = b0_ref[...] + b1_ref[...]
    o_ref[...] = lax.dot_general(
        a, b, (((1,), (1,)), ((), ())),
        preferred_element_type=jnp.float32)


def _tc_decoder(z0, z1, rblk=200):
    npad, k = z0.shape
    n = N_NODES
    return pl.pallas_call(
        _decoder_body,
        grid=(n // rblk,),
        in_specs=[
            pl.BlockSpec((rblk, k), lambda i: (i, 0)),
            pl.BlockSpec((rblk, k), lambda i: (i, 0)),
            pl.BlockSpec((n, k), lambda i: (0, 0)),
            pl.BlockSpec((n, k), lambda i: (0, 0)),
        ],
        out_specs=pl.BlockSpec((rblk, n), lambda i: (i, 0)),
        out_shape=jax.ShapeDtypeStruct((n, n), jnp.float32),
    )(z0, z1, z0, z1)


CHUNK = 96
NROWS = 3
NIDX = 4


def _segsum_body(table, src_h, dst_h, ew_h, zero_h, out,
                 acc, srcv, dstv, ewv, rows, src_t, dst_t, ew_t, rows_t,
                 sem_src, sem_dst, sem_ew, sem_g, sem_s,
                 *, edge_split, scale_vecs):
    c = lax.axis_index("c")
    s = lax.axis_index("s")
    npt = NPAD // NS
    C = CHUNK

    if edge_split:
        ept = N_EDGES // (NC * NS)
        ebase = (c * NS + s) * ept
        tbl = table
    else:
        ept = N_EDGES // NS
        ebase = s * ept
        tbl = table.at[c]
    nch = ept // C
    tail = ept - nch * C

    pltpu.sync_copy(zero_h, acc.at[pl.ds(s * npt, npt)])
    plsc.subcore_barrier()

    def issue_idx(k):
        buf = lax.rem(k, NIDX)
        pltpu.async_copy(src_h.at[pl.ds(ebase + k * C, C)],
                         srcv.at[buf], sem_src.at[buf])
        pltpu.async_copy(dst_h.at[pl.ds(ebase + k * C, C)],
                         dstv.at[buf], sem_dst.at[buf])
        pltpu.async_copy(ew_h.at[pl.ds(ebase + k * C, C)],
                         ewv.at[buf], sem_ew.at[buf])

    def wait_src(k):
        buf = lax.rem(k, NIDX)
        pltpu.make_async_copy(src_h.at[pl.ds(ebase, C)],
                              srcv.at[buf], sem_src.at[buf]).wait()

    def wait_dst(k):
        buf = lax.rem(k, NIDX)
        pltpu.make_async_copy(dst_h.at[pl.ds(ebase, C)],
                              dstv.at[buf], sem_dst.at[buf]).wait()

    def wait_ew(k):
        buf = lax.rem(k, NIDX)
        pltpu.make_async_copy(ew_h.at[pl.ds(ebase, C)],
                              ewv.at[buf], sem_ew.at[buf]).wait()

    def issue_gather(k):
        rb = lax.rem(k, NROWS)
        ib = lax.rem(k, NIDX)
        pltpu.async_copy(tbl.at[srcv.at[ib]], rows.at[rb], sem_g.at[rb])

    def wait_gather(k):
        rb = lax.rem(k, NROWS)
        pltpu.make_async_copy(tbl.at[srcv.at[0]], rows.at[rb],
                              sem_g.at[rb]).wait()

    def issue_scatter(k):
        rb = lax.rem(k, NROWS)
        ib = lax.rem(k, NIDX)
        pltpu.async_copy(rows.at[rb], acc.at[dstv.at[ib]],
                         sem_s.at[rb], add=True)

    def wait_scatter(k):
        rb = lax.rem(k, NROWS)
        ib = lax.rem(k, NIDX)
        pltpu.make_async_copy(rows.at[rb], acc.at[dstv.at[ib]],
                              sem_s.at[rb]).wait()

    for kk in range(min(3, nch)):
        issue_idx(kk)
    if nch > 0:
        wait_src(0)
        issue_gather(0)
    if nch > 1:
        wait_src(1)
        issue_gather(1)

    def step(k, carry):
        ib = lax.rem(k, NIDX)
        rb = lax.rem(k, NROWS)
        wait_gather(k)

        @pl.when(k >= 1)
        def _drain():
            wait_scatter(k - 1)

        @pl.when(k + 3 < nch)
        def _prefetch_idx():
            issue_idx(k + 3)

        @pl.when(k + 2 < nch)
        def _prefetch_rows():
            wait_src(k + 2)
            issue_gather(k + 2)

        wait_ew(k)

        @plsc.parallel_loop(0, C // L, unroll=2)
        def _scale(g):
            wv = ewv[ib, pl.ds(g * L, L)]
            for j in range(L):
                e = g * L + j
                w = jnp.full((L,), wv[j], jnp.float32)
                for f in range(scale_vecs):
                    sl = pl.ds(f * L, L)
                    rows[rb, e, sl] = rows[rb, e, sl] * w
        wait_dst(k)
        issue_scatter(k)
        return carry

    lax.fori_loop(0, nch, step, 0)

    if tail:
        tb = nch * C
        pltpu.sync_copy(src_h.at[pl.ds(ebase + tb, tail)], src_t)
        pltpu.sync_copy(dst_h.at[pl.ds(ebase + tb, tail)], dst_t)
        pltpu.sync_copy(ew_h.at[pl.ds(ebase + tb, tail)],
                        ew_t.at[pl.ds(0, tail)])
        pltpu.sync_copy(tbl.at[src_t], rows_t)
        wv = ew_t[...]
        for j in range(tail):
            w = jnp.full((L,), wv[j], jnp.float32)
            for f in range(scale_vecs):
                sl = pl.ds(f * L, L)
                rows_t[j, sl] = rows_t[j, sl] * w
        pltpu.sync_copy(rows_t, acc.at[dst_t], add=True)

    wait_scatter(nch - 1)
    plsc.subcore_barrier()

    pltpu.sync_copy(acc.at[pl.ds(s * npt, npt)],
                    out.at[c].at[pl.ds(s * npt, npt)])


def _make_segsum(*, edge_split, scale_vecs, name):
    body = functools.partial(
        _segsum_body, edge_split=edge_split, scale_vecs=scale_vecs)
    mesh = plsc.VectorSubcoreMesh(core_axis_name="c", subcore_axis_name="s")
    ept = N_EDGES // (NC * NS) if edge_split else N_EDGES // NS
    tail = ept % CHUNK
    return pl.kernel(
        body,
        out_type=jax.ShapeDtypeStruct((NC, NPAD, FW), jnp.float32),
        mesh=mesh,
        scratch_types=[
            pltpu.VMEM_SHARED((NPAD, FW), jnp.float32),
            pltpu.VMEM((NIDX, CHUNK), jnp.int32),
            pltpu.VMEM((NIDX, CHUNK), jnp.int32),
            pltpu.VMEM((NIDX, CHUNK), jnp.float32),
            pltpu.VMEM((NROWS, CHUNK, FW), jnp.float32),
            pltpu.VMEM((max(tail, 1),), jnp.int32),
            pltpu.VMEM((max(tail, 1),), jnp.int32),
            pltpu.VMEM((L,), jnp.float32),
            pltpu.VMEM((max(tail, 1), FW), jnp.float32),
            pltpu.SemaphoreType.DMA((NIDX,)),
            pltpu.SemaphoreType.DMA((NIDX,)),
            pltpu.SemaphoreType.DMA((NIDX,)),
            pltpu.SemaphoreType.DMA((NROWS,)),
            pltpu.SemaphoreType.DMA((NROWS,)),
        ],
        name=name,
    )


_segsum_l1 = _make_segsum(edge_split=False, scale_vecs=FW // L,
                          name="segsum_l1")
_segsum_l2 = _make_segsum(edge_split=True, scale_vecs=D_H2 // L,
                          name="segsum_l2")


@jax.jit
def kernel(x, edge_index, edge_weight, W1, W2):
    src = edge_index[0]
    dst = edge_index[1]
    zeros = jnp.zeros((NPAD // NS, FW), jnp.float32)

    hw1 = _tc_matmul(x, W1)
    hw1_split = hw1.reshape(N_NODES, NC, D_H1 // NC).transpose(1, 0, 2)
    h1_pre = _segsum_l1(hw1_split, src, dst, edge_weight, zeros)

    w2_split = jnp.pad(W2.reshape(NC, D_H1 // NC, D_H2),
                       ((0, 0), (0, 0), (0, FW - D_H2)))
    hz = _tc_relu_matmul(h1_pre, w2_split)
    z_parts = _segsum_l2(hz, src, dst, edge_weight, zeros)

    recon = _tc_decoder(z_parts[0], z_parts[1])
    return recon.reshape(-1)


# speedup vs baseline: 3.3588x; 1.2060x over previous
import functools

import jax
import jax.numpy as jnp
from jax import lax
from jax.experimental import pallas as pl
from jax.experimental.pallas import tpu as pltpu
from jax.experimental.pallas import tpu_sc as plsc

N_NODES = 10000
N_EDGES = 160000
D_IN = 256
D_H1 = 256
D_H2 = 64

NC = 2
NS = 16
L = 16
FW = 128
NPAD = 10240


def _mm_body(x_ref, w_ref, o_ref):
    o_ref[...] = jnp.dot(x_ref[...], w_ref[...],
                         preferred_element_type=jnp.float32)


def _tc_matmul(x, w, blk=1000):
    n, k = x.shape
    m = w.shape[1]
    return pl.pallas_call(
        _mm_body,
        grid=(n // blk,),
        in_specs=[
            pl.BlockSpec((blk, k), lambda i: (i, 0)),
            pl.BlockSpec((k, m), lambda i: (0, 0)),
        ],
        out_specs=pl.BlockSpec((blk, m), lambda i: (i, 0)),
        out_shape=jax.ShapeDtypeStruct((n, m), jnp.float32),
    )(x, w)


def _relu_mm_body(h_ref, w_ref, o_ref):
    a = jnp.maximum(h_ref[0], 0.0)
    b = jnp.maximum(h_ref[1], 0.0)
    o_ref[...] = (jnp.dot(a, w_ref[0], preferred_element_type=jnp.float32)
                  + jnp.dot(b, w_ref[1], preferred_element_type=jnp.float32))


def _tc_relu_matmul(h_split, w_split, blk=1024):
    _, n, k = h_split.shape
    m = w_split.shape[2]
    return pl.pallas_call(
        _relu_mm_body,
        grid=(n // blk,),
        in_specs=[
            pl.BlockSpec((2, blk, k), lambda i: (0, i, 0)),
            pl.BlockSpec((2, k, m), lambda i: (0, 0, 0)),
        ],
        out_specs=pl.BlockSpec((blk, m), lambda i: (i, 0)),
        out_shape=jax.ShapeDtypeStruct((n, m), jnp.float32),
    )(h_split, w_split)


def _decoder_body(a0_ref, a1_ref, b0_ref, b1_ref, o_ref):
    a = a0_ref[...] + a1_ref[...]
    b = b0_ref[...] + b1_ref[...]
    o_ref[...] = lax.dot_general(
        a, b, (((1,), (1,)), ((), ())),
        preferred_element_type=jnp.float32)


def _tc_decoder(z0, z1, rblk=200):
    npad, k = z0.shape
    n = N_NODES
    return pl.pallas_call(
        _decoder_body,
        grid=(n // rblk,),
        in_specs=[
            pl.BlockSpec((rblk, k), lambda i: (i, 0)),
            pl.BlockSpec((rblk, k), lambda i: (i, 0)),
            pl.BlockSpec((n, k), lambda i: (0, 0)),
            pl.BlockSpec((n, k), lambda i: (0, 0)),
        ],
        out_specs=pl.BlockSpec((rblk, n), lambda i: (i, 0)),
        out_shape=jax.ShapeDtypeStruct((n, n), jnp.float32),
    )(z0, z1, z0, z1)


CHUNK = 80
NROWS = 4
NIDX = 4


def _segsum_body(table, src_h, dst_h, ew_h, zero_h, out,
                 acc, srcv, dstv, ewv, rows, src_t, dst_t, ew_t, rows_t,
                 sem_src, sem_dst, sem_ew, sem_g, sem_s,
                 *, edge_split, scale_vecs):
    c = lax.axis_index("c")
    s = lax.axis_index("s")
    npt = NPAD // NS
    C = CHUNK

    if edge_split:
        ept = N_EDGES // (NC * NS)
        ebase = (c * NS + s) * ept
        tbl = table
    else:
        ept = N_EDGES // NS
        ebase = s * ept
        tbl = table.at[c]
    nch = ept // C
    tail = ept - nch * C

    pltpu.sync_copy(zero_h, acc.at[pl.ds(s * npt, npt)])
    plsc.subcore_barrier()

    def issue_idx(k):
        buf = lax.rem(k, NIDX)
        pltpu.async_copy(src_h.at[pl.ds(ebase + k * C, C)],
                         srcv.at[buf], sem_src.at[buf])
        pltpu.async_copy(dst_h.at[pl.ds(ebase + k * C, C)],
                         dstv.at[buf], sem_dst.at[buf])
        pltpu.async_copy(ew_h.at[pl.ds(ebase + k * C, C)],
                         ewv.at[buf], sem_ew.at[buf])

    def wait_src(k):
        buf = lax.rem(k, NIDX)
        pltpu.make_async_copy(src_h.at[pl.ds(ebase, C)],
                              srcv.at[buf], sem_src.at[buf]).wait()

    def wait_dst(k):
        buf = lax.rem(k, NIDX)
        pltpu.make_async_copy(dst_h.at[pl.ds(ebase, C)],
                              dstv.at[buf], sem_dst.at[buf]).wait()

    def wait_ew(k):
        buf = lax.rem(k, NIDX)
        pltpu.make_async_copy(ew_h.at[pl.ds(ebase, C)],
                              ewv.at[buf], sem_ew.at[buf]).wait()

    def issue_gather(k):
        rb = lax.rem(k, NROWS)
        ib = lax.rem(k, NIDX)
        pltpu.async_copy(tbl.at[srcv.at[ib]], rows.at[rb], sem_g.at[rb])

    def wait_gather(k):
        rb = lax.rem(k, NROWS)
        pltpu.make_async_copy(tbl.at[srcv.at[0]], rows.at[rb],
                              sem_g.at[rb]).wait()

    def issue_scatter(k):
        rb = lax.rem(k, NROWS)
        ib = lax.rem(k, NIDX)
        pltpu.async_copy(rows.at[rb], acc.at[dstv.at[ib]],
                         sem_s.at[rb], add=True)

    def wait_scatter(k):
        rb = lax.rem(k, NROWS)
        ib = lax.rem(k, NIDX)
        pltpu.make_async_copy(rows.at[rb], acc.at[dstv.at[ib]],
                              sem_s.at[rb]).wait()

    for kk in range(min(3, nch)):
        issue_idx(kk)
    if nch > 0:
        wait_src(0)
        issue_gather(0)
    if nch > 1:
        wait_src(1)
        issue_gather(1)

    def step(k, carry):
        ib = lax.rem(k, NIDX)
        rb = lax.rem(k, NROWS)
        wait_gather(k)

        @pl.when(k >= 1)
        def _drain():
            wait_scatter(k - 1)

        @pl.when(k + 3 < nch)
        def _prefetch_idx():
            issue_idx(k + 3)

        @pl.when(k + 2 < nch)
        def _prefetch_rows():
            wait_src(k + 2)
            issue_gather(k + 2)

        wait_ew(k)

        @plsc.parallel_loop(0, C // L, unroll=C // L)
        def _scale(g):
            wv = ewv[ib, pl.ds(g * L, L)]
            for j in range(L):
                e = g * L + j
                w = jnp.full((L,), wv[j], jnp.float32)
                for f in range(scale_vecs):
                    sl = pl.ds(f * L, L)
                    rows[rb, e, sl] = rows[rb, e, sl] * w
        wait_dst(k)
        issue_scatter(k)
        return carry

    lax.fori_loop(0, nch, step, 0)

    if tail:
        tb = nch * C
        pltpu.sync_copy(src_h.at[pl.ds(ebase + tb, tail)], src_t)
        pltpu.sync_copy(dst_h.at[pl.ds(ebase + tb, tail)], dst_t)
        pltpu.sync_copy(ew_h.at[pl.ds(ebase + tb, tail)],
                        ew_t.at[pl.ds(0, tail)])
        pltpu.sync_copy(tbl.at[src_t], rows_t)
        for g0 in range(0, tail, L):
            wv = ew_t[pl.ds(g0, L)]
            for j in range(min(L, tail - g0)):
                e = g0 + j
                w = jnp.full((L,), wv[j], jnp.float32)
                for f in range(scale_vecs):
                    sl = pl.ds(f * L, L)
                    rows_t[e, sl] = rows_t[e, sl] * w
        pltpu.sync_copy(rows_t, acc.at[dst_t], add=True)

    wait_scatter(nch - 1)
    plsc.subcore_barrier()

    pltpu.sync_copy(acc.at[pl.ds(s * npt, npt)],
                    out.at[c].at[pl.ds(s * npt, npt)])


def _make_segsum(*, edge_split, scale_vecs, name):
    body = functools.partial(
        _segsum_body, edge_split=edge_split, scale_vecs=scale_vecs)
    mesh = plsc.VectorSubcoreMesh(core_axis_name="c", subcore_axis_name="s")
    ept = N_EDGES // (NC * NS) if edge_split else N_EDGES // NS
    tail = ept % CHUNK
    return pl.kernel(
        body,
        out_type=jax.ShapeDtypeStruct((NC, NPAD, FW), jnp.float32),
        mesh=mesh,
        scratch_types=[
            pltpu.VMEM_SHARED((NPAD, FW), jnp.float32),
            pltpu.VMEM((NIDX, CHUNK), jnp.int32),
            pltpu.VMEM((NIDX, CHUNK), jnp.int32),
            pltpu.VMEM((NIDX, CHUNK), jnp.float32),
            pltpu.VMEM((NROWS, CHUNK, FW), jnp.float32),
            pltpu.VMEM((max(tail, 1),), jnp.int32),
            pltpu.VMEM((max(tail, 1),), jnp.int32),
            pltpu.VMEM((((max(tail, 1) + L - 1) // L) * L,), jnp.float32),
            pltpu.VMEM((max(tail, 1), FW), jnp.float32),
            pltpu.SemaphoreType.DMA((NIDX,)),
            pltpu.SemaphoreType.DMA((NIDX,)),
            pltpu.SemaphoreType.DMA((NIDX,)),
            pltpu.SemaphoreType.DMA((NROWS,)),
            pltpu.SemaphoreType.DMA((NROWS,)),
        ],
        name=name,
    )


_segsum_l1 = _make_segsum(edge_split=False, scale_vecs=FW // L,
                          name="segsum_l1")
_segsum_l2 = _make_segsum(edge_split=True, scale_vecs=D_H2 // L,
                          name="segsum_l2")


@jax.jit
def kernel(x, edge_index, edge_weight, W1, W2):
    src = edge_index[0]
    dst = edge_index[1]
    zeros = jnp.zeros((NPAD // NS, FW), jnp.float32)

    hw1 = _tc_matmul(x, W1)
    hw1_split = hw1.reshape(N_NODES, NC, D_H1 // NC).transpose(1, 0, 2)
    h1_pre = _segsum_l1(hw1_split, src, dst, edge_weight, zeros)

    w2_split = jnp.pad(W2.reshape(NC, D_H1 // NC, D_H2),
                       ((0, 0), (0, 0), (0, FW - D_H2)))
    hz = _tc_relu_matmul(h1_pre, w2_split)
    z_parts = _segsum_l2(hz, src, dst, edge_weight, zeros)

    recon = _tc_decoder(z_parts[0], z_parts[1])
    return recon.reshape(-1)


# speedup vs baseline: 3.3674x; 1.0026x over previous
import functools

import jax
import jax.numpy as jnp
from jax import lax
from jax.experimental import pallas as pl
from jax.experimental.pallas import tpu as pltpu
from jax.experimental.pallas import tpu_sc as plsc

N_NODES = 10000
N_EDGES = 160000
D_IN = 256
D_H1 = 256
D_H2 = 64

NC = 2
NS = 16
L = 16
FW = 128
NPAD = 10240


def _mm_body(x_ref, w_ref, o_ref):
    o_ref[...] = jnp.dot(x_ref[...], w_ref[...],
                         preferred_element_type=jnp.float32)


def _tc_matmul(x, w, blk=1000):
    n, k = x.shape
    m = w.shape[1]
    return pl.pallas_call(
        _mm_body,
        grid=(n // blk,),
        in_specs=[
            pl.BlockSpec((blk, k), lambda i: (i, 0)),
            pl.BlockSpec((k, m), lambda i: (0, 0)),
        ],
        out_specs=pl.BlockSpec((blk, m), lambda i: (i, 0)),
        out_shape=jax.ShapeDtypeStruct((n, m), jnp.float32),
    )(x, w)


def _relu_mm_body(h_ref, w_ref, o_ref):
    a = jnp.maximum(h_ref[0], 0.0)
    b = jnp.maximum(h_ref[1], 0.0)
    o_ref[...] = (jnp.dot(a, w_ref[0], preferred_element_type=jnp.float32)
                  + jnp.dot(b, w_ref[1], preferred_element_type=jnp.float32))


def _tc_relu_matmul(h_split, w_split, blk=1024):
    _, n, k = h_split.shape
    m = w_split.shape[2]
    return pl.pallas_call(
        _relu_mm_body,
        grid=(n // blk,),
        in_specs=[
            pl.BlockSpec((2, blk, k), lambda i: (0, i, 0)),
            pl.BlockSpec((2, k, m), lambda i: (0, 0, 0)),
        ],
        out_specs=pl.BlockSpec((blk, m), lambda i: (i, 0)),
        out_shape=jax.ShapeDtypeStruct((n, m), jnp.float32),
    )(h_split, w_split)


def _decoder_body(a0_ref, a1_ref, b0_ref, b1_ref, o_ref):
    a = a0_ref[...] + a1_ref[...]
    b = b0_ref[...] + b1_ref[...]
    o_ref[...] = lax.dot_general(
        a, b, (((1,), (1,)), ((), ())),
        preferred_element_type=jnp.float32)


def _tc_decoder(z0, z1, rblk=400):
    npad, k = z0.shape
    n = N_NODES
    return pl.pallas_call(
        _decoder_body,
        grid=(n // rblk,),
        in_specs=[
            pl.BlockSpec((rblk, k), lambda i: (i, 0)),
            pl.BlockSpec((rblk, k), lambda i: (i, 0)),
            pl.BlockSpec((n, k), lambda i: (0, 0)),
            pl.BlockSpec((n, k), lambda i: (0, 0)),
        ],
        out_specs=pl.BlockSpec((rblk, n), lambda i: (i, 0)),
        out_shape=jax.ShapeDtypeStruct((n, n), jnp.float32),
    )(z0, z1, z0, z1)


CHUNK = 80
NROWS = 4
NIDX = 4


def _segsum_body(table, src_h, dst_h, ew_h, zero_h, out,
                 acc, srcv, dstv, ewv, rows, src_t, dst_t, ew_t, rows_t,
                 sem_src, sem_dst, sem_ew, sem_g, sem_s,
                 *, edge_split, scale_vecs):
    c = lax.axis_index("c")
    s = lax.axis_index("s")
    npt = NPAD // NS
    C = CHUNK

    if edge_split:
        ept = N_EDGES // (NC * NS)
        ebase = (c * NS + s) * ept
        tbl = table
    else:
        ept = N_EDGES // NS
        ebase = s * ept
        tbl = table.at[c]
    nch = ept // C
    tail = ept - nch * C

    pltpu.sync_copy(zero_h, acc.at[pl.ds(s * npt, npt)])
    plsc.subcore_barrier()

    def issue_idx(k):
        buf = lax.rem(k, NIDX)
        pltpu.async_copy(src_h.at[pl.ds(ebase + k * C, C)],
                         srcv.at[buf], sem_src.at[buf])
        pltpu.async_copy(dst_h.at[pl.ds(ebase + k * C, C)],
                         dstv.at[buf], sem_dst.at[buf])
        pltpu.async_copy(ew_h.at[pl.ds(ebase + k * C, C)],
                         ewv.at[buf], sem_ew.at[buf])

    def wait_src(k):
        buf = lax.rem(k, NIDX)
        pltpu.make_async_copy(src_h.at[pl.ds(ebase, C)],
                              srcv.at[buf], sem_src.at[buf]).wait()

    def wait_dst(k):
        buf = lax.rem(k, NIDX)
        pltpu.make_async_copy(dst_h.at[pl.ds(ebase, C)],
                              dstv.at[buf], sem_dst.at[buf]).wait()

    def wait_ew(k):
        buf = lax.rem(k, NIDX)
        pltpu.make_async_copy(ew_h.at[pl.ds(ebase, C)],
                              ewv.at[buf], sem_ew.at[buf]).wait()

    def issue_gather(k):
        rb = lax.rem(k, NROWS)
        ib = lax.rem(k, NIDX)
        pltpu.async_copy(tbl.at[srcv.at[ib]], rows.at[rb], sem_g.at[rb])

    def wait_gather(k):
        rb = lax.rem(k, NROWS)
        pltpu.make_async_copy(tbl.at[srcv.at[0]], rows.at[rb],
                              sem_g.at[rb]).wait()

    def issue_scatter(k):
        rb = lax.rem(k, NROWS)
        ib = lax.rem(k, NIDX)
        pltpu.async_copy(rows.at[rb], acc.at[dstv.at[ib]],
                         sem_s.at[rb], add=True)

    def wait_scatter(k):
        rb = lax.rem(k, NROWS)
        ib = lax.rem(k, NIDX)
        pltpu.make_async_copy(rows.at[rb], acc.at[dstv.at[ib]],
                              sem_s.at[rb]).wait()

    for kk in range(min(3, nch)):
        issue_idx(kk)
    if nch > 0:
        wait_src(0)
        issue_gather(0)
    if nch > 1:
        wait_src(1)
        issue_gather(1)

    def step(k, carry):
        ib = lax.rem(k, NIDX)
        rb = lax.rem(k, NROWS)
        wait_gather(k)

        @pl.when(k >= 1)
        def _drain():
            wait_scatter(k - 1)

        @pl.when(k + 3 < nch)
        def _prefetch_idx():
            issue_idx(k + 3)

        @pl.when(k + 2 < nch)
        def _prefetch_rows():
            wait_src(k + 2)
            issue_gather(k + 2)

        wait_ew(k)

        @plsc.parallel_loop(0, C // L, unroll=C // L)
        def _scale(g):
            wv = ewv[ib, pl.ds(g * L, L)]
            for j in range(L):
                e = g * L + j
                w = jnp.full((L,), wv[j], jnp.float32)
                for f in range(scale_vecs):
                    sl = pl.ds(f * L, L)
                    rows[rb, e, sl] = rows[rb, e, sl] * w
        wait_dst(k)
        issue_scatter(k)
        return carry

    lax.fori_loop(0, nch, step, 0)

    if tail:
        tb = nch * C
        pltpu.sync_copy(src_h.at[pl.ds(ebase + tb, tail)], src_t)
        pltpu.sync_copy(dst_h.at[pl.ds(ebase + tb, tail)], dst_t)
        pltpu.sync_copy(ew_h.at[pl.ds(ebase + tb, tail)],
                        ew_t.at[pl.ds(0, tail)])
        pltpu.sync_copy(tbl.at[src_t], rows_t)
        for g0 in range(0, tail, L):
            wv = ew_t[pl.ds(g0, L)]
            for j in range(min(L, tail - g0)):
                e = g0 + j
                w = jnp.full((L,), wv[j], jnp.float32)
                for f in range(scale_vecs):
                    sl = pl.ds(f * L, L)
                    rows_t[e, sl] = rows_t[e, sl] * w
        pltpu.sync_copy(rows_t, acc.at[dst_t], add=True)

    wait_scatter(nch - 1)
    plsc.subcore_barrier()

    pltpu.sync_copy(acc.at[pl.ds(s * npt, npt)],
                    out.at[c].at[pl.ds(s * npt, npt)])


def _make_segsum(*, edge_split, scale_vecs, name):
    body = functools.partial(
        _segsum_body, edge_split=edge_split, scale_vecs=scale_vecs)
    mesh = plsc.VectorSubcoreMesh(core_axis_name="c", subcore_axis_name="s")
    ept = N_EDGES // (NC * NS) if edge_split else N_EDGES // NS
    tail = ept % CHUNK
    return pl.kernel(
        body,
        out_type=jax.ShapeDtypeStruct((NC, NPAD, FW), jnp.float32),
        mesh=mesh,
        scratch_types=[
            pltpu.VMEM_SHARED((NPAD, FW), jnp.float32),
            pltpu.VMEM((NIDX, CHUNK), jnp.int32),
            pltpu.VMEM((NIDX, CHUNK), jnp.int32),
            pltpu.VMEM((NIDX, CHUNK), jnp.float32),
            pltpu.VMEM((NROWS, CHUNK, FW), jnp.float32),
            pltpu.VMEM((max(tail, 1),), jnp.int32),
            pltpu.VMEM((max(tail, 1),), jnp.int32),
            pltpu.VMEM((((max(tail, 1) + L - 1) // L) * L,), jnp.float32),
            pltpu.VMEM((max(tail, 1), FW), jnp.float32),
            pltpu.SemaphoreType.DMA((NIDX,)),
            pltpu.SemaphoreType.DMA((NIDX,)),
            pltpu.SemaphoreType.DMA((NIDX,)),
            pltpu.SemaphoreType.DMA((NROWS,)),
            pltpu.SemaphoreType.DMA((NROWS,)),
        ],
        name=name,
    )


_segsum_l1 = _make_segsum(edge_split=False, scale_vecs=FW // L,
                          name="segsum_l1")
_segsum_l2 = _make_segsum(edge_split=True, scale_vecs=D_H2 // L,
                          name="segsum_l2")


@jax.jit
def kernel(x, edge_index, edge_weight, W1, W2):
    src = edge_index[0]
    dst = edge_index[1]
    zeros = jnp.zeros((NPAD // NS, FW), jnp.float32)

    hw1 = _tc_matmul(x, W1)
    hw1_split = hw1.reshape(N_NODES, NC, D_H1 // NC).transpose(1, 0, 2)
    h1_pre = _segsum_l1(hw1_split, src, dst, edge_weight, zeros)

    w2_split = jnp.pad(W2.reshape(NC, D_H1 // NC, D_H2),
                       ((0, 0), (0, 0), (0, FW - D_H2)))
    hz = _tc_relu_matmul(h1_pre, w2_split)
    z_parts = _segsum_l2(hz, src, dst, edge_weight, zeros)

    recon = _tc_decoder(z_parts[0], z_parts[1])
    return recon.reshape(-1)
